# Initial kernel scaffold; baseline (speedup 1.0000x reference)
import jax
import jax.numpy as jnp
from jax.experimental import pallas as pl


def kernel(x, dists_max, dists_argmax, W_h1, b_h1, W_p1, b_p1, W_h2, b_h2, W_p2, b_p2, W_o, b_o):
    raise NotImplementedError("write your pallas kernel here")


# speedup vs baseline: 1.7880x; 1.7880x over previous
import functools

import jax
import jax.numpy as jnp
from jax import lax
from jax.experimental import pallas as pl
from jax.experimental.pallas import tpu as pltpu
from jax.experimental.pallas import tpu_sc as plsc

N_NODES = 10000
A = 32
D = 128

NC = 2
NS = 16
NW = NC * NS
N_PAD = 10240
NPW = N_PAD // NW
SB = 32
NSB = NPW // SB
BLK = 8
NSUB = SB // BLK
KC = D // 16


def _lin_body(x_ref, wt_ref, wb_ref, b_ref, y_ref, s_ref, *, prescale):
    xb = x_ref[...]
    if prescale is not None:
        xb = jnp.maximum(xb * prescale, 0.0)
    y_ref[...] = jnp.dot(xb, wt_ref[...], preferred_element_type=jnp.float32,
                       precision=lax.Precision.HIGHEST)
    s_ref[...] = (jnp.dot(xb, wb_ref[...], preferred_element_type=jnp.float32,
                       precision=lax.Precision.HIGHEST)
                  + b_ref[...])


def _linear(x, W, b, prescale):
    wt, wb = W[:D], W[D:]
    b2 = b.reshape(1, D)
    grid = (N_PAD // 1024,)
    return pl.pallas_call(
        functools.partial(_lin_body, prescale=prescale),
        grid=grid,
        in_specs=[
            pl.BlockSpec((1024, D), lambda i: (i, 0)),
            pl.BlockSpec((D, D), lambda i: (0, 0)),
            pl.BlockSpec((D, D), lambda i: (0, 0)),
            pl.BlockSpec((1, D), lambda i: (0, 0)),
        ],
        out_specs=[
            pl.BlockSpec((1024, D), lambda i: (i, 0)),
            pl.BlockSpec((1024, D), lambda i: (i, 0)),
        ],
        out_shape=[
            jax.ShapeDtypeStruct((N_PAD, D), jnp.float32),
            jax.ShapeDtypeStruct((N_PAD, D), jnp.float32),
        ],
    )(x, wt, wb, b2)


def _fin_body(pp_ref, wo_ref, bo_ref, o_ref):
    pp = pp_ref[...]
    grp = lax.broadcasted_iota(jnp.int32, (A * 16, A), 0) // 16
    col = lax.broadcasted_iota(jnp.int32, (A * 16, A), 1)
    m = (grp == col).astype(jnp.float32)
    p = jnp.dot(pp, m, preferred_element_type=jnp.float32,
                       precision=lax.Precision.HIGHEST)
    nrm = jnp.maximum(jnp.sqrt(jnp.sum(p * p, axis=1, keepdims=True)), 1e-12)
    o_ref[...] = (jnp.dot(p / nrm, wo_ref[...],
                          preferred_element_type=jnp.float32,
                       precision=lax.Precision.HIGHEST) + bo_ref[...])


def _finish(pp, W_o, b_o):
    grid = (N_PAD // 1024,)
    return pl.pallas_call(
        _fin_body,
        grid=grid,
        in_specs=[
            pl.BlockSpec((1024, A * 16), lambda i: (i, 0)),
            pl.BlockSpec((A, D), lambda i: (0, 0)),
            pl.BlockSpec((1, D), lambda i: (0, 0)),
        ],
        out_specs=pl.BlockSpec((1024, D), lambda i: (i, 0)),
        out_shape=jax.ShapeDtypeStruct((N_PAD, D), jnp.float32),
    )(pp, W_o, b_o.reshape(1, D))


_MESH = plsc.VectorSubcoreMesh(core_axis_name="c", subcore_axis_name="s")


def _worker_id():
    return lax.axis_index("s") * NC + lax.axis_index("c")


@functools.partial(
    pl.kernel,
    mesh=_MESH,
    out_type=jax.ShapeDtypeStruct((N_PAD, D), jnp.float32),
    scratch_types=[
        pltpu.VMEM((SB * A // 128, 128), jnp.int32),
        pltpu.VMEM((SB * A + 16,), jnp.float32),
        pltpu.VMEM((BLK * A, D), jnp.float32),
        pltpu.VMEM((SB, D), jnp.float32),
        pltpu.VMEM((SB, D), jnp.float32),
        pltpu.SemaphoreType.DMA,
    ],
)
def _sc_pass1(y_hbm, s_hbm, d_hbm, idx_hbm, out_hbm,
              idx_v, d_v, rows_v, s_v, out_v, sem):
    wid = _worker_id()

    def block_fn(b, carry):
        node0 = pl.multiple_of(wid * NPW + b * SB, SB)
        pltpu.sync_copy(
            idx_hbm.at[pl.ds(pl.multiple_of(node0 * A // 128, SB * A // 128),
                             SB * A // 128)],
            idx_v)
        pltpu.sync_copy(d_hbm.at[pl.ds(node0 * A, SB * A)],
                        d_v.at[pl.ds(0, SB * A)])
        pltpu.sync_copy(s_hbm.at[pl.ds(node0, SB)], s_v)
        for sb in range(NSUB):
            cps = [pltpu.async_copy(y_hbm.at[idx_v.at[sb * 2 + j]],
                                    rows_v.at[pl.ds(j * 128, 128)], sem)
                   for j in range(2)]
            for cp in cps:
                cp.wait()
            for i in range(BLK):
                ni = sb * BLK + i
                s_ch = [s_v[ni, pl.ds(kc * 16, 16)] for kc in range(KC)]

                def anchor_fn(a, acc):
                    dvec = d_v[pl.ds(ni * A + a, 16)][0]
                    r = i * A + a
                    new = []
                    for kc in range(KC):
                        row = rows_v[r, pl.ds(kc * 16, 16)]
                        new.append(acc[kc]
                                   + jnp.maximum(dvec * row + s_ch[kc], 0.0))
                    return tuple(new)

                acc = lax.fori_loop(
                    0, A, anchor_fn,
                    tuple(jnp.zeros((16,), jnp.float32) for _ in range(KC)))
                for kc in range(KC):
                    out_v[ni, pl.ds(kc * 16, 16)] = acc[kc]
        pltpu.sync_copy(out_v, out_hbm.at[pl.ds(node0, SB)])
        return carry

    lax.fori_loop(0, NSB, block_fn, 0)


@functools.partial(
    pl.kernel,
    mesh=_MESH,
    out_type=jax.ShapeDtypeStruct((N_PAD, A * 16), jnp.float32),
    scratch_types=[
        pltpu.VMEM((SB * A // 128, 128), jnp.int32),
        pltpu.VMEM((SB * A + 16,), jnp.float32),
        pltpu.VMEM((BLK * A, D), jnp.float32),
        pltpu.VMEM((SB, D), jnp.float32),
        pltpu.VMEM((D,), jnp.float32),
        pltpu.VMEM((16,), jnp.float32),
        pltpu.VMEM((SB, A * 16), jnp.float32),
        pltpu.SemaphoreType.DMA,
    ],
)
def _sc_pass2(y_hbm, s_hbm, d_hbm, idx_hbm, wp_hbm, bp_hbm, out_hbm,
              idx_v, d_v, rows_v, s_v, wp_v, bp_v, out_v, sem):
    wid = _worker_id()
    pltpu.sync_copy(wp_hbm, wp_v)
    pltpu.sync_copy(bp_hbm, bp_v)

    def block_fn(b, carry):
        node0 = pl.multiple_of(wid * NPW + b * SB, SB)
        pltpu.sync_copy(
            idx_hbm.at[pl.ds(pl.multiple_of(node0 * A // 128, SB * A // 128),
                             SB * A // 128)],
            idx_v)
        pltpu.sync_copy(d_hbm.at[pl.ds(node0 * A, SB * A)],
                        d_v.at[pl.ds(0, SB * A)])
        pltpu.sync_copy(s_hbm.at[pl.ds(node0, SB)], s_v)
        bp_vec = bp_v[...]
        wp_ch = [wp_v[pl.ds(kc * 16, 16)] for kc in range(KC)]
        for sb in range(NSUB):
            cps = [pltpu.async_copy(y_hbm.at[idx_v.at[sb * 2 + j]],
                                    rows_v.at[pl.ds(j * 128, 128)], sem)
                   for j in range(2)]
            for cp in cps:
                cp.wait()
            for i in range(BLK):
                ni = sb * BLK + i
                s_ch = [s_v[ni, pl.ds(kc * 16, 16)] for kc in range(KC)]

                def anchor_fn(a, carry2):
                    dvec = d_v[pl.ds(ni * A + a, 16)][0]
                    r = i * A + a
                    acc = bp_vec
                    for kc in range(KC):
                        row = rows_v[r, pl.ds(kc * 16, 16)]
                        acc = acc + (jnp.maximum(dvec * row + s_ch[kc],
                                                 0.0) * wp_ch[kc])
                    out_v[ni, pl.ds(a * 16, 16)] = acc
                    return carry2

                lax.fori_loop(0, A, anchor_fn, 0)
        pltpu.sync_copy(out_v, out_hbm.at[pl.ds(node0, SB)])
        return carry

    lax.fori_loop(0, NSB, block_fn, 0)


def kernel(x, dists_max, dists_argmax,
           W_h1, b_h1, W_p1, b_p1, W_h2, b_h2, W_p2, b_p2, W_o, b_o):
    pad = N_PAD - N_NODES
    x_p = jnp.pad(x, ((0, pad), (0, 0)))
    d_p = jnp.pad(dists_max, ((0, pad), (0, 0))).reshape(N_PAD * A)
    idx_p = jnp.pad(dists_argmax.astype(jnp.int32), ((0, pad), (0, 0)))
    idx_flat = idx_p.reshape(N_PAD * A // 128, 128)

    y1, s1 = _linear(x_p, W_h1, b_h1, prescale=None)
    sum1 = _sc_pass1(y1, s1, d_p, idx_flat)
    y2, s2 = _linear(sum1, W_h2, b_h2, prescale=1.0 / A)
    bp_init = jnp.full((16,), b_p2[0] / 16.0, jnp.float32)
    p = _sc_pass2(y2, s2, d_p, idx_flat, W_p2[:, 0], bp_init)
    out = _finish(p, W_o, b_o)
    return out[:N_NODES]


# speedup vs baseline: 1.8616x; 1.0412x over previous
import functools

import jax
import jax.numpy as jnp
import numpy as np
from jax import lax
from jax.experimental import pallas as pl
from jax.experimental.pallas import tpu as pltpu
from jax.experimental.pallas import tpu_sc as plsc

N_NODES = 10000
A = 32
D = 128

NC = 2
NS = 16
NW = NC * NS
N_PAD = 10240
NPW = N_PAD // NW
SB = 32
NSB = NPW // SB
BLK = 8
NSUB = SB // BLK
KC2 = D // 32

_pos = np.arange(D)
_PERM = (_pos // 32) * 32 + 2 * (_pos % 16) + (_pos % 32) // 16


def _lin_body(x_ref, wt_ref, wb_ref, b_ref, y_ref, s_ref, *, prescale):
    xb = x_ref[...]
    if prescale is not None:
        xb = jnp.maximum(xb * prescale, 0.0)
    y_ref[...] = jnp.dot(xb, wt_ref[...], preferred_element_type=jnp.float32,
                         precision=lax.Precision.HIGHEST).astype(jnp.bfloat16)
    s_ref[...] = (jnp.dot(xb, wb_ref[...], preferred_element_type=jnp.float32,
                          precision=lax.Precision.HIGHEST) + b_ref[...])


def _linear(x, wt, wb, b, prescale):
    grid = (N_PAD // 1024,)
    return pl.pallas_call(
        functools.partial(_lin_body, prescale=prescale),
        grid=grid,
        in_specs=[
            pl.BlockSpec((1024, D), lambda i: (i, 0)),
            pl.BlockSpec((D, D), lambda i: (0, 0)),
            pl.BlockSpec((D, D), lambda i: (0, 0)),
            pl.BlockSpec((1, D), lambda i: (0, 0)),
        ],
        out_specs=[
            pl.BlockSpec((1024, D), lambda i: (i, 0)),
            pl.BlockSpec((1024, D), lambda i: (i, 0)),
        ],
        out_shape=[
            jax.ShapeDtypeStruct((N_PAD, D), jnp.bfloat16),
            jax.ShapeDtypeStruct((N_PAD, D), jnp.float32),
        ],
    )(x, wt, wb, b.reshape(1, D))


def _fin_body(pp_ref, wo_ref, bo_ref, o_ref):
    pp = pp_ref[...]
    grp = lax.broadcasted_iota(jnp.int32, (A * 16, A), 0) // 16
    col = lax.broadcasted_iota(jnp.int32, (A * 16, A), 1)
    m = (grp == col).astype(jnp.float32)
    p = jnp.dot(pp, m, preferred_element_type=jnp.float32,
                precision=lax.Precision.HIGHEST)
    nrm = jnp.maximum(jnp.sqrt(jnp.sum(p * p, axis=1, keepdims=True)), 1e-12)
    o_ref[...] = (jnp.dot(p / nrm, wo_ref[...],
                          preferred_element_type=jnp.float32,
                          precision=lax.Precision.HIGHEST) + bo_ref[...])


def _finish(pp, W_o, b_o):
    grid = (N_PAD // 1024,)
    return pl.pallas_call(
        _fin_body,
        grid=grid,
        in_specs=[
            pl.BlockSpec((1024, A * 16), lambda i: (i, 0)),
            pl.BlockSpec((A, D), lambda i: (0, 0)),
            pl.BlockSpec((1, D), lambda i: (0, 0)),
        ],
        out_specs=pl.BlockSpec((1024, D), lambda i: (i, 0)),
        out_shape=jax.ShapeDtypeStruct((N_PAD, D), jnp.float32),
    )(pp, W_o, b_o.reshape(1, D))


_MESH = plsc.VectorSubcoreMesh(core_axis_name="c", subcore_axis_name="s")
_HI_MASK = np.int32(-65536)


def _worker_id():
    return lax.axis_index("s") * NC + lax.axis_index("c")


def _row_halves(w):
    lo = lax.bitcast_convert_type(jnp.left_shift(w, 16), jnp.float32)
    hi = lax.bitcast_convert_type(w & _HI_MASK, jnp.float32)
    return lo, hi


@functools.partial(
    pl.kernel,
    mesh=_MESH,
    out_type=jax.ShapeDtypeStruct((N_PAD, D), jnp.float32),
    compiler_params=pltpu.CompilerParams(use_tc_tiling_on_sc=False),
    scratch_types=[
        pltpu.VMEM((SB * A // 128, 128), jnp.int32),
        pltpu.VMEM((SB * A + 16,), jnp.float32),
        pltpu.VMEM((2, BLK * A, D // 2), jnp.int32),
        pltpu.VMEM((SB, D), jnp.float32),
        pltpu.VMEM((SB, D), jnp.float32),
        pltpu.SemaphoreType.DMA,
        pltpu.SemaphoreType.DMA,
    ],
)
def _sc_pass1(y_hbm, s_hbm, d_hbm, idx_hbm, out_hbm,
              idx_v, d_v, rows_v, s_v, out_v, sem0, sem1):
    wid = _worker_id()
    sems = [sem0, sem1]

    def gathers(sb):
        buf = sb % 2
        return [pltpu.async_copy(y_hbm.at[idx_v.at[sb * 2 + j]],
                                 rows_v.at[buf].at[pl.ds(j * 128, 128)],
                                 sems[buf])
                for j in range(2)]

    def block_fn(b, carry):
        node0 = pl.multiple_of(wid * NPW + b * SB, SB)
        pltpu.sync_copy(
            idx_hbm.at[pl.ds(pl.multiple_of(node0 * A // 128, SB * A // 128),
                             SB * A // 128)],
            idx_v)
        pltpu.sync_copy(d_hbm.at[pl.ds(node0 * A, SB * A)],
                        d_v.at[pl.ds(0, SB * A)])
        pltpu.sync_copy(s_hbm.at[pl.ds(node0, SB)], s_v)
        pend = gathers(0)
        for sb in range(NSUB):
            buf = sb % 2
            nxt = gathers(sb + 1) if sb + 1 < NSUB else []
            for cp in pend:
                cp.wait()
            pend = nxt
            for i in range(BLK):
                ni = sb * BLK + i
                s_ch = [s_v[ni, pl.ds(k * 16, 16)] for k in range(2 * KC2)]
                zeros = tuple(jnp.zeros((16,), jnp.float32)
                              for _ in range(2 * KC2))

                @plsc.parallel_loop(0, A, unroll=2, carry=zeros)
                def acc(a, c):
                    dvec = d_v[pl.ds(ni * A + a, 16)][0]
                    new = []
                    for k in range(KC2):
                        lo, hi = _row_halves(
                            rows_v[buf, i * A + a, pl.ds(k * 16, 16)])
                        new.append(c[2 * k] + jnp.maximum(
                            dvec * lo + s_ch[2 * k], 0.0))
                        new.append(c[2 * k + 1] + jnp.maximum(
                            dvec * hi + s_ch[2 * k + 1], 0.0))
                    return tuple(new)

                for k in range(2 * KC2):
                    out_v[ni, pl.ds(k * 16, 16)] = acc[k]
        pltpu.sync_copy(out_v, out_hbm.at[pl.ds(node0, SB)])
        return carry

    lax.fori_loop(0, NSB, block_fn, 0)


@functools.partial(
    pl.kernel,
    mesh=_MESH,
    out_type=jax.ShapeDtypeStruct((N_PAD, A * 16), jnp.float32),
    compiler_params=pltpu.CompilerParams(use_tc_tiling_on_sc=False),
    scratch_types=[
        pltpu.VMEM((SB * A // 128, 128), jnp.int32),
        pltpu.VMEM((SB * A + 16,), jnp.float32),
        pltpu.VMEM((2, BLK * A, D // 2), jnp.int32),
        pltpu.VMEM((SB, D), jnp.float32),
        pltpu.VMEM((D,), jnp.float32),
        pltpu.VMEM((16,), jnp.float32),
        pltpu.VMEM((SB, A * 16), jnp.float32),
        pltpu.SemaphoreType.DMA,
        pltpu.SemaphoreType.DMA,
    ],
)
def _sc_pass2(y_hbm, s_hbm, d_hbm, idx_hbm, wp_hbm, bp_hbm, out_hbm,
              idx_v, d_v, rows_v, s_v, wp_v, bp_v, out_v, sem0, sem1):
    wid = _worker_id()
    sems = [sem0, sem1]
    pltpu.sync_copy(wp_hbm, wp_v)
    pltpu.sync_copy(bp_hbm, bp_v)

    def gathers(sb):
        buf = sb % 2
        return [pltpu.async_copy(y_hbm.at[idx_v.at[sb * 2 + j]],
                                 rows_v.at[buf].at[pl.ds(j * 128, 128)],
                                 sems[buf])
                for j in range(2)]

    def block_fn(b, carry):
        node0 = pl.multiple_of(wid * NPW + b * SB, SB)
        pltpu.sync_copy(
            idx_hbm.at[pl.ds(pl.multiple_of(node0 * A // 128, SB * A // 128),
                             SB * A // 128)],
            idx_v)
        pltpu.sync_copy(d_hbm.at[pl.ds(node0 * A, SB * A)],
                        d_v.at[pl.ds(0, SB * A)])
        pltpu.sync_copy(s_hbm.at[pl.ds(node0, SB)], s_v)
        bp_vec = bp_v[...]
        wp_ch = [wp_v[pl.ds(k * 16, 16)] for k in range(2 * KC2)]
        pend = gathers(0)
        for sb in range(NSUB):
            buf = sb % 2
            nxt = gathers(sb + 1) if sb + 1 < NSUB else []
            for cp in pend:
                cp.wait()
            pend = nxt
            for i in range(BLK):
                ni = sb * BLK + i
                s_ch = [s_v[ni, pl.ds(k * 16, 16)] for k in range(2 * KC2)]

                @plsc.parallel_loop(0, A, unroll=2)
                def _(a):
                    dvec = d_v[pl.ds(ni * A + a, 16)][0]
                    acc = bp_vec
                    for k in range(KC2):
                        lo, hi = _row_halves(
                            rows_v[buf, i * A + a, pl.ds(k * 16, 16)])
                        acc = acc + (jnp.maximum(dvec * lo + s_ch[2 * k],
                                                 0.0) * wp_ch[2 * k])
                        acc = acc + (jnp.maximum(dvec * hi + s_ch[2 * k + 1],
                                                 0.0) * wp_ch[2 * k + 1])
                    out_v[ni, pl.ds(a * 16, 16)] = acc
        pltpu.sync_copy(out_v, out_hbm.at[pl.ds(node0, SB)])
        return carry

    lax.fori_loop(0, NSB, block_fn, 0)


def kernel(x, dists_max, dists_argmax,
           W_h1, b_h1, W_p1, b_p1, W_h2, b_h2, W_p2, b_p2, W_o, b_o):
    pad = N_PAD - N_NODES
    x_p = jnp.pad(x, ((0, pad), (0, 0)))
    d_p = jnp.pad(dists_max, ((0, pad), (0, 0))).reshape(N_PAD * A)
    idx_p = jnp.pad(dists_argmax.astype(jnp.int32), ((0, pad), (0, 0)))
    idx_flat = idx_p.reshape(N_PAD * A // 128, 128)
    perm = _PERM

    y1, s1 = _linear(x_p, W_h1[:D], W_h1[D:, perm], b_h1[perm], None)
    y1i = lax.bitcast_convert_type(y1.reshape(N_PAD, D // 2, 2), jnp.int32)
    h_perm = _sc_pass1(y1i, s1, d_p, idx_flat)

    y2, s2 = _linear(h_perm, W_h2[:D][perm, :], W_h2[D:][perm][:, perm],
                     b_h2[perm], 1.0 / A)
    y2i = lax.bitcast_convert_type(y2.reshape(N_PAD, D // 2, 2), jnp.int32)
    bp_init = jnp.full((16,), b_p2[0] / 16.0, jnp.float32)
    pp = _sc_pass2(y2i, s2, d_p, idx_flat, W_p2[perm, 0], bp_init)
    out = _finish(pp, W_o, b_o)
    return out[:N_NODES]


# speedup vs baseline: 2.0594x; 1.1062x over previous
import functools

import jax
import jax.numpy as jnp
import numpy as np
from jax import lax
from jax.experimental import pallas as pl
from jax.experimental.pallas import tpu as pltpu
from jax.experimental.pallas import tpu_sc as plsc

N_NODES = 10000
A = 32
D = 128

NC = 2
NS = 16
NW = NC * NS
N_PAD = 10240
NPW = N_PAD // NW
SB = 32
NSB = NPW // SB
BLK = 4
NSUB = SB // BLK
KC = D // 16


def _lin_body(x_ref, wt_ref, wb_ref, b_ref, y_ref, s_ref, *, prescale):
    xb = x_ref[...]
    if prescale is not None:
        xb = jnp.maximum(xb * prescale, 0.0)
    y_ref[...] = jnp.dot(xb, wt_ref[...], preferred_element_type=jnp.float32,
                         precision=lax.Precision.HIGHEST)
    s_ref[...] = (jnp.dot(xb, wb_ref[...], preferred_element_type=jnp.float32,
                          precision=lax.Precision.HIGHEST) + b_ref[...])


def _linear(x, wt, wb, b, prescale):
    grid = (N_PAD // 1024,)
    return pl.pallas_call(
        functools.partial(_lin_body, prescale=prescale),
        grid=grid,
        in_specs=[
            pl.BlockSpec((1024, D), lambda i: (i, 0)),
            pl.BlockSpec((D, D), lambda i: (0, 0)),
            pl.BlockSpec((D, D), lambda i: (0, 0)),
            pl.BlockSpec((1, D), lambda i: (0, 0)),
        ],
        out_specs=[
            pl.BlockSpec((1024, D), lambda i: (i, 0)),
            pl.BlockSpec((1024, D), lambda i: (i, 0)),
        ],
        out_shape=[
            jax.ShapeDtypeStruct((N_PAD, D), jnp.float32),
            jax.ShapeDtypeStruct((N_PAD, D), jnp.float32),
        ],
    )(x, wt, wb, b.reshape(1, D))


def _fin_body(pp_ref, wo_ref, bo_ref, o_ref):
    pp = pp_ref[...]
    grp = lax.broadcasted_iota(jnp.int32, (A * 16, A), 0) // 16
    col = lax.broadcasted_iota(jnp.int32, (A * 16, A), 1)
    m = (grp == col).astype(jnp.float32)
    p = jnp.dot(pp, m, preferred_element_type=jnp.float32,
                precision=lax.Precision.HIGHEST)
    nrm = jnp.maximum(jnp.sqrt(jnp.sum(p * p, axis=1, keepdims=True)), 1e-12)
    o_ref[...] = (jnp.dot(p / nrm, wo_ref[...],
                          preferred_element_type=jnp.float32,
                          precision=lax.Precision.HIGHEST) + bo_ref[...])


def _finish(pp, W_o, b_o):
    grid = (N_PAD // 1024,)
    return pl.pallas_call(
        _fin_body,
        grid=grid,
        in_specs=[
            pl.BlockSpec((1024, A * 16), lambda i: (i, 0)),
            pl.BlockSpec((A, D), lambda i: (0, 0)),
            pl.BlockSpec((1, D), lambda i: (0, 0)),
        ],
        out_specs=pl.BlockSpec((1024, D), lambda i: (i, 0)),
        out_shape=jax.ShapeDtypeStruct((N_PAD, D), jnp.float32),
    )(pp, W_o, b_o.reshape(1, D))


_MESH = plsc.VectorSubcoreMesh(core_axis_name="c", subcore_axis_name="s")


def _worker_id():
    return lax.axis_index("s") * NC + lax.axis_index("c")


@functools.partial(
    pl.kernel,
    mesh=_MESH,
    out_type=jax.ShapeDtypeStruct((N_PAD, D), jnp.float32),
    compiler_params=pltpu.CompilerParams(use_tc_tiling_on_sc=False),
    scratch_types=[
        pltpu.VMEM((SB * A // 128, 128), jnp.int32),
        pltpu.VMEM((SB * A + 16,), jnp.float32),
        pltpu.VMEM((2, BLK * A, D), jnp.float32),
        pltpu.VMEM((SB, D), jnp.float32),
        pltpu.VMEM((SB, D), jnp.float32),
        pltpu.SemaphoreType.DMA,
        pltpu.SemaphoreType.DMA,
    ],
)
def _sc_pass1(y_hbm, s_hbm, d_hbm, idx_hbm, out_hbm,
              idx_v, d_v, rows_v, s_v, out_v, sem0, sem1):
    wid = _worker_id()
    sems = [sem0, sem1]

    def gather(sb):
        buf = sb % 2
        return pltpu.async_copy(y_hbm.at[idx_v.at[sb]], rows_v.at[buf],
                                sems[buf])

    def block_fn(b, carry):
        node0 = pl.multiple_of(wid * NPW + b * SB, SB)
        pltpu.sync_copy(
            idx_hbm.at[pl.ds(pl.multiple_of(node0 * A // 128, SB * A // 128),
                             SB * A // 128)],
            idx_v)
        pltpu.sync_copy(d_hbm.at[pl.ds(node0 * A, SB * A)],
                        d_v.at[pl.ds(0, SB * A)])
        pltpu.sync_copy(s_hbm.at[pl.ds(node0, SB)], s_v)
        pend = gather(0)
        for sb in range(NSUB):
            buf = sb % 2
            nxt = gather(sb + 1) if sb + 1 < NSUB else None
            pend.wait()
            pend = nxt
            for i in range(BLK):
                ni = sb * BLK + i
                s_ch = [s_v[ni, pl.ds(k * 16, 16)] for k in range(KC)]
                zeros = tuple(jnp.zeros((16,), jnp.float32)
                              for _ in range(KC))

                @plsc.parallel_loop(0, A, unroll=2, carry=zeros)
                def acc(a, c):
                    dvec = d_v[pl.ds(ni * A + a, 16)][0]
                    return tuple(
                        c[k] + jnp.maximum(
                            dvec * rows_v[buf, i * A + a, pl.ds(k * 16, 16)]
                            + s_ch[k], 0.0)
                        for k in range(KC))

                for k in range(KC):
                    out_v[ni, pl.ds(k * 16, 16)] = acc[k]
        pltpu.sync_copy(out_v, out_hbm.at[pl.ds(node0, SB)])
        return carry

    lax.fori_loop(0, NSB, block_fn, 0)


@functools.partial(
    pl.kernel,
    mesh=_MESH,
    out_type=jax.ShapeDtypeStruct((N_PAD, A * 16), jnp.float32),
    compiler_params=pltpu.CompilerParams(use_tc_tiling_on_sc=False),
    scratch_types=[
        pltpu.VMEM((SB * A // 128, 128), jnp.int32),
        pltpu.VMEM((SB * A + 16,), jnp.float32),
        pltpu.VMEM((2, BLK * A, D), jnp.float32),
        pltpu.VMEM((SB, D), jnp.float32),
        pltpu.VMEM((D,), jnp.float32),
        pltpu.VMEM((16,), jnp.float32),
        pltpu.VMEM((SB, A * 16), jnp.float32),
        pltpu.SemaphoreType.DMA,
        pltpu.SemaphoreType.DMA,
    ],
)
def _sc_pass2(y_hbm, s_hbm, d_hbm, idx_hbm, wp_hbm, bp_hbm, out_hbm,
              idx_v, d_v, rows_v, s_v, wp_v, bp_v, out_v, sem0, sem1):
    wid = _worker_id()
    sems = [sem0, sem1]
    pltpu.sync_copy(wp_hbm, wp_v)
    pltpu.sync_copy(bp_hbm, bp_v)

    def gather(sb):
        buf = sb % 2
        return pltpu.async_copy(y_hbm.at[idx_v.at[sb]], rows_v.at[buf],
                                sems[buf])

    def block_fn(b, carry):
        node0 = pl.multiple_of(wid * NPW + b * SB, SB)
        pltpu.sync_copy(
            idx_hbm.at[pl.ds(pl.multiple_of(node0 * A // 128, SB * A // 128),
                             SB * A // 128)],
            idx_v)
        pltpu.sync_copy(d_hbm.at[pl.ds(node0 * A, SB * A)],
                        d_v.at[pl.ds(0, SB * A)])
        pltpu.sync_copy(s_hbm.at[pl.ds(node0, SB)], s_v)
        bp_vec = bp_v[...]
        wp_ch = [wp_v[pl.ds(k * 16, 16)] for k in range(KC)]
        pend = gather(0)
        for sb in range(NSUB):
            buf = sb % 2
            nxt = gather(sb + 1) if sb + 1 < NSUB else None
            pend.wait()
            pend = nxt
            for i in range(BLK):
                ni = sb * BLK + i
                s_ch = [s_v[ni, pl.ds(k * 16, 16)] for k in range(KC)]

                @plsc.parallel_loop(0, A, unroll=2)
                def _(a):
                    dvec = d_v[pl.ds(ni * A + a, 16)][0]
                    acc = bp_vec
                    for k in range(KC):
                        acc = acc + (jnp.maximum(
                            dvec * rows_v[buf, i * A + a, pl.ds(k * 16, 16)]
                            + s_ch[k], 0.0) * wp_ch[k])
                    out_v[ni, pl.ds(a * 16, 16)] = acc
        pltpu.sync_copy(out_v, out_hbm.at[pl.ds(node0, SB)])
        return carry

    lax.fori_loop(0, NSB, block_fn, 0)


def kernel(x, dists_max, dists_argmax,
           W_h1, b_h1, W_p1, b_p1, W_h2, b_h2, W_p2, b_p2, W_o, b_o):
    pad = N_PAD - N_NODES
    x_p = jnp.pad(x, ((0, pad), (0, 0)))
    d_p = jnp.pad(dists_max, ((0, pad), (0, 0))).reshape(N_PAD * A)
    idx_p = jnp.pad(dists_argmax.astype(jnp.int32), ((0, pad), (0, 0)))
    idx_flat = idx_p.reshape(N_PAD * A // 128, 128)

    y1, s1 = _linear(x_p, W_h1[:D], W_h1[D:], b_h1, None)
    h = _sc_pass1(y1, s1, d_p, idx_flat)

    y2, s2 = _linear(h, W_h2[:D], W_h2[D:], b_h2, 1.0 / A)
    bp_init = jnp.full((16,), b_p2[0] / 16.0, jnp.float32)
    pp = _sc_pass2(y2, s2, d_p, idx_flat, W_p2[:, 0], bp_init)
    out = _finish(pp, W_o, b_o)
    return out[:N_NODES]
